# Initial kernel scaffold; baseline (speedup 1.0000x reference)
import jax
import jax.numpy as jnp
from jax.experimental import pallas as pl


def kernel(x, embed, G_indices, G_values, W1, b1, W2, b2, ln1_scale, ln1_bias, ln2_scale, ln2_bias):
    raise NotImplementedError("write your pallas kernel here")


# speedup vs baseline: 3.8714x; 3.8714x over previous
import functools

import jax
import jax.numpy as jnp
from jax import lax
from jax.experimental import pallas as pl
from jax.experimental.pallas import tpu as pltpu
from jax.experimental.pallas import tpu_sc as plsc

_N = 10000
_E = 320000
_D = 128
_B = 16384

_NC = 2
_NS = 16
_L = 16
_NW = _NC * _NS
_EPT = _E // _NW
_K = 80
_NCHUNK = _EPT // _K
_RPT = 624
_RTAIL = _N - _NS * _RPT

_sc_mesh = plsc.VectorSubcoreMesh(core_axis_name="c", subcore_axis_name="s")


@functools.partial(
    pl.kernel,
    out_type=jax.ShapeDtypeStruct((_NC, _N, _D), jnp.float32),
    mesh=_sc_mesh,
    scratch_types=[
        pltpu.VMEM((_K,), jnp.int32),
        pltpu.VMEM((_K,), jnp.int32),
        pltpu.VMEM((_K,), jnp.float32),
        pltpu.VMEM((_K, _D), jnp.float32),
        pltpu.VMEM_SHARED((_N, _D), jnp.float32),
        pltpu.SemaphoreType.DMA,
    ],
)
def _spmm_sc(e_hbm, row_hbm, col_hbm, val_hbm, zeros_hbm, out_hbm,
             colv, rowv, valv, rows, acc, sem):
    s = lax.axis_index("s")
    c = lax.axis_index("c")
    w = s * _NC + c
    pltpu.sync_copy(zeros_hbm.at[pl.ds(s * _RPT, _RPT)],
                    acc.at[pl.ds(s * _RPT, _RPT)])

    @pl.when(s == 0)
    def _():
        pltpu.sync_copy(zeros_hbm.at[pl.ds(_NS * _RPT, _RTAIL)],
                        acc.at[pl.ds(_NS * _RPT, _RTAIL)])

    plsc.subcore_barrier()

    ebase = w * _EPT

    def chunk_body(j, carry):
        off = ebase + j * _K
        pltpu.sync_copy(col_hbm.at[pl.ds(off, _K)], colv)
        pltpu.sync_copy(row_hbm.at[pl.ds(off, _K)], rowv)
        pltpu.sync_copy(val_hbm.at[pl.ds(off, _K)], valv)
        pltpu.async_copy(e_hbm.at[colv], rows, sem).wait()
        for g in range(_K // _L):
            v16 = valv[pl.ds(g * _L, _L)]
            for ll in range(_L):
                sp = jnp.broadcast_to(v16[ll], (_L,))
                e_i = g * _L + ll
                for d in range(_D // _L):
                    sl = pl.ds(d * _L, _L)
                    rows[e_i, sl] = rows[e_i, sl] * sp
        pltpu.sync_copy(rows, acc.at[rowv], add=True)
        return carry

    lax.fori_loop(0, _NCHUNK, chunk_body, 0)
    plsc.subcore_barrier()
    pltpu.sync_copy(acc.at[pl.ds(s * _RPT, _RPT)],
                    out_hbm.at[c, pl.ds(s * _RPT, _RPT)])

    @pl.when(s == 0)
    def _():
        pltpu.sync_copy(acc.at[pl.ds(_NS * _RPT, _RTAIL)],
                        out_hbm.at[c, pl.ds(_NS * _RPT, _RTAIL)])


_BPT = _B // _NW
_GK = 128
_GCH = _BPT // _GK


@functools.partial(
    pl.kernel,
    out_type=jax.ShapeDtypeStruct((_B, _D), jnp.float32),
    mesh=_sc_mesh,
    scratch_types=[
        pltpu.VMEM((_GK,), jnp.int32),
        pltpu.VMEM((_GK,), jnp.int32),
        pltpu.VMEM((_GK, _D), jnp.float32),
        pltpu.SemaphoreType.DMA,
    ],
)
def _lookup_sc(t_hbm, x_hbm, out_hbm, xv, idxv, rows, sem):
    s = lax.axis_index("s")
    c = lax.axis_index("c")
    w = s * _NC + c
    base = w * _BPT

    def body(j, carry):
        off = base + j * _GK
        pltpu.sync_copy(x_hbm.at[pl.ds(off, _GK)], xv)
        for g in range(_GK // _L):
            sl = pl.ds(g * _L, _L)
            x16 = xv[sl]
            idxv[sl] = jnp.where(x16 >= 1, x16 - 1, _N)
        pltpu.async_copy(t_hbm.at[idxv], rows, sem).wait()
        pltpu.sync_copy(rows, out_hbm.at[pl.ds(off, _GK)])
        return carry

    lax.fori_loop(0, _GCH, body, 0)


def _mm1_body(e_ref, w_ref, b_ref, o_ref):
    o_ref[...] = (jnp.dot(e_ref[...], w_ref[...],
                          preferred_element_type=jnp.float32) + b_ref[...])


_mm1 = pl.pallas_call(
    _mm1_body,
    out_shape=jax.ShapeDtypeStruct((_N, _D), jnp.float32),
)


def _layer_norm(r, g, b):
    mu = jnp.mean(r, axis=-1, keepdims=True)
    var = jnp.mean((r - mu) ** 2, axis=-1, keepdims=True)
    return (r - mu) * lax.rsqrt(var + 1e-5) * g + b


def _mid_body(p_ref, w_ref, b_ref, g_ref, be_ref, e1_ref, h2_ref):
    r = jnp.maximum(p_ref[0] + p_ref[1], 0.0)
    e1 = _layer_norm(r, g_ref[...], be_ref[...])
    e1_ref[...] = e1
    h2_ref[...] = (jnp.dot(e1, w_ref[...],
                           preferred_element_type=jnp.float32) + b_ref[...])


_mid = pl.pallas_call(
    _mid_body,
    out_shape=(jax.ShapeDtypeStruct((_N, _D), jnp.float32),
               jax.ShapeDtypeStruct((_N, _D), jnp.float32)),
)


def _fin_body(p_ref, e1_ref, g_ref, be_ref, o_ref):
    r = jnp.maximum(p_ref[0] + p_ref[1], 0.0)
    e = _layer_norm(r, g_ref[...], be_ref[...]) + e1_ref[...]
    o_ref[pl.ds(0, _N), :] = e
    o_ref[pl.ds(_N, 8), :] = jnp.zeros((8, _D), jnp.float32)


_fin = pl.pallas_call(
    _fin_body,
    out_shape=jax.ShapeDtypeStruct((_N + 8, _D), jnp.float32),
)


def kernel(x, embed, G_indices, G_values, W1, b1, W2, b2,
           ln1_scale, ln1_bias, ln2_scale, ln2_bias):
    row = G_indices[0]
    col = G_indices[1]
    zeros_nd = jnp.zeros((_N, _D), jnp.float32)
    b1r = b1.reshape(1, _D)
    b2r = b2.reshape(1, _D)
    g1 = ln1_scale.reshape(1, _D)
    a1 = ln1_bias.reshape(1, _D)
    g2 = ln2_scale.reshape(1, _D)
    a2 = ln2_bias.reshape(1, _D)

    h1 = _mm1(embed.astype(jnp.float32), W1, b1r)
    p1 = _spmm_sc(h1, row, col, G_values, zeros_nd)
    e1, h2 = _mid(p1, W2, b2r, g1, a1)
    p2 = _spmm_sc(h2, row, col, G_values, zeros_nd)
    t = _fin(p2, e1, g2, a2)
    final = _lookup_sc(t, x.astype(jnp.int32))
    recon_loss = jnp.zeros((1,), dtype=jnp.float32)
    return (final, recon_loss)


# speedup vs baseline: 7.7722x; 2.0076x over previous
import functools

import jax
import jax.numpy as jnp
from jax import lax
from jax.experimental import pallas as pl
from jax.experimental.pallas import tpu as pltpu
from jax.experimental.pallas import tpu_sc as plsc

_N = 10000
_E = 320000
_D = 128
_B = 16384

_NC = 2
_NS = 16
_L = 16
_NW = _NC * _NS
_DH = _D // 2
_EPT = _E // _NS
_K = 80
_NCHUNK = _EPT // _K
_RPT = 624
_RTAIL = _N - _NS * _RPT

_sc_mesh = plsc.VectorSubcoreMesh(core_axis_name="c", subcore_axis_name="s")


_NBUF = 5


@functools.partial(
    pl.kernel,
    out_type=jax.ShapeDtypeStruct((_NC, _N, _DH), jnp.float32),
    mesh=_sc_mesh,
    scratch_types=[
        pltpu.VMEM((_NCHUNK, _K), jnp.int32),
        pltpu.VMEM((_NCHUNK, _K), jnp.int32),
        pltpu.VMEM((_NCHUNK, _K), jnp.float32),
        pltpu.VMEM((_NBUF, _K, _DH), jnp.float32),
        pltpu.VMEM_SHARED((_N, _DH), jnp.float32),
        pltpu.SemaphoreType.DMA((_NBUF,)),
    ],
    compiler_params=pltpu.CompilerParams(use_tc_tiling_on_sc=False),
)
def _spmm_sc(e2_hbm, row_hbm, colA_hbm, colB_hbm, val_hbm, zeros_hbm, out_hbm,
             colv, rowv, valv, rows, acc, sem):
    s = lax.axis_index("s")
    c = lax.axis_index("c")
    @pl.when(c == 0)
    def _():
        pltpu.sync_copy(colA_hbm.at[s], colv)

    @pl.when(c == 1)
    def _():
        pltpu.sync_copy(colB_hbm.at[s], colv)

    pltpu.sync_copy(row_hbm.at[s], rowv)
    pltpu.sync_copy(val_hbm.at[s], valv)
    for b in range(_NBUF):
        pltpu.async_copy(e2_hbm.at[colv.at[b]], rows.at[b], sem.at[b])
    pltpu.sync_copy(zeros_hbm.at[pl.ds(s * _RPT, _RPT)],
                    acc.at[pl.ds(s * _RPT, _RPT)])

    @pl.when(s == 0)
    def _():
        pltpu.sync_copy(zeros_hbm.at[pl.ds(_NS * _RPT, _RTAIL)],
                        acc.at[pl.ds(_NS * _RPT, _RTAIL)])

    plsc.subcore_barrier()

    def outer_body(jo, carry):
        for b in range(_NBUF):
            j = jo * _NBUF + b
            pltpu.make_async_copy(e2_hbm.at[colv.at[j]], rows.at[b],
                                  sem.at[b]).wait()
            for g in range(_K // _L):
                v16 = valv[j, pl.ds(g * _L, _L)]
                for ll in range(_L):
                    sp = jnp.broadcast_to(v16[ll], (_L,))
                    e_i = g * _L + ll
                    for d in range(_DH // _L):
                        sl = pl.ds(d * _L, _L)
                        rows[b, e_i, sl] = rows[b, e_i, sl] * sp
            pltpu.sync_copy(rows.at[b], acc.at[rowv.at[j]], add=True)

            @pl.when(j + _NBUF < _NCHUNK)
            def _():
                pltpu.async_copy(e2_hbm.at[colv.at[j + _NBUF]], rows.at[b],
                                 sem.at[b])

        return carry

    lax.fori_loop(0, _NCHUNK // _NBUF, outer_body, 0)
    plsc.subcore_barrier()
    pltpu.sync_copy(acc.at[pl.ds(s * _RPT, _RPT)],
                    out_hbm.at[c, pl.ds(s * _RPT, _RPT)])

    @pl.when(s == 0)
    def _():
        pltpu.sync_copy(acc.at[pl.ds(_NS * _RPT, _RTAIL)],
                        out_hbm.at[c, pl.ds(_NS * _RPT, _RTAIL)])


_BPT = _B // _NW
_GK = 128
_GCH = _BPT // _GK


@functools.partial(
    pl.kernel,
    out_type=jax.ShapeDtypeStruct((_B, _D), jnp.float32),
    mesh=_sc_mesh,
    scratch_types=[
        pltpu.VMEM((_GK,), jnp.int32),
        pltpu.VMEM((_GK,), jnp.int32),
        pltpu.VMEM((_GK, _D), jnp.float32),
        pltpu.SemaphoreType.DMA,
    ],
)
def _lookup_sc(t_hbm, x_hbm, out_hbm, xv, idxv, rows, sem):
    s = lax.axis_index("s")
    c = lax.axis_index("c")
    w = s * _NC + c
    base = w * _BPT

    def body(j, carry):
        off = base + j * _GK
        pltpu.sync_copy(x_hbm.at[pl.ds(off, _GK)], xv)
        for g in range(_GK // _L):
            sl = pl.ds(g * _L, _L)
            x16 = xv[sl]
            idxv[sl] = jnp.where(x16 >= 1, x16 - 1, _N)
        pltpu.async_copy(t_hbm.at[idxv], rows, sem).wait()
        pltpu.sync_copy(rows, out_hbm.at[pl.ds(off, _GK)])
        return carry

    lax.fori_loop(0, _GCH, body, 0)


def _mm1_body(e_ref, w_ref, b_ref, o_ref):
    o_ref[...] = (jnp.dot(e_ref[...], w_ref[...],
                          preferred_element_type=jnp.float32) + b_ref[...])


_mm1 = pl.pallas_call(
    _mm1_body,
    out_shape=jax.ShapeDtypeStruct((_N, _D), jnp.float32),
)


def _layer_norm(r, g, b):
    mu = jnp.mean(r, axis=-1, keepdims=True)
    var = jnp.mean((r - mu) ** 2, axis=-1, keepdims=True)
    return (r - mu) * lax.rsqrt(var + 1e-5) * g + b


def _mid_body(p_ref, w_ref, b_ref, g_ref, be_ref, e1_ref, h2_ref):
    r = jnp.maximum(jnp.concatenate([p_ref[0], p_ref[1]], axis=-1), 0.0)
    e1 = _layer_norm(r, g_ref[...], be_ref[...])
    e1_ref[...] = e1
    h2_ref[...] = (jnp.dot(e1, w_ref[...],
                           preferred_element_type=jnp.float32) + b_ref[...])


_mid = pl.pallas_call(
    _mid_body,
    out_shape=(jax.ShapeDtypeStruct((_N, _D), jnp.float32),
               jax.ShapeDtypeStruct((_N, _D), jnp.float32)),
)


def _fin_body(p_ref, e1_ref, g_ref, be_ref, o_ref):
    r = jnp.maximum(jnp.concatenate([p_ref[0], p_ref[1]], axis=-1), 0.0)
    e = _layer_norm(r, g_ref[...], be_ref[...]) + e1_ref[...]
    o_ref[pl.ds(0, _N), :] = e
    o_ref[pl.ds(_N, 8), :] = jnp.zeros((8, _D), jnp.float32)


_fin = pl.pallas_call(
    _fin_body,
    out_shape=jax.ShapeDtypeStruct((_N + 8, _D), jnp.float32),
)


def kernel(x, embed, G_indices, G_values, W1, b1, W2, b2,
           ln1_scale, ln1_bias, ln2_scale, ln2_bias):
    row = G_indices[0].reshape(_NS, _NCHUNK, _K)
    colA = (G_indices[1] * 2).reshape(_NS, _NCHUNK, _K)
    colB = (G_indices[1] * 2 + 1).reshape(_NS, _NCHUNK, _K)
    vals = G_values.reshape(_NS, _NCHUNK, _K)
    zeros_nd = jnp.zeros((_N, _DH), jnp.float32)
    b1r = b1.reshape(1, _D)
    b2r = b2.reshape(1, _D)
    g1 = ln1_scale.reshape(1, _D)
    a1 = ln1_bias.reshape(1, _D)
    g2 = ln2_scale.reshape(1, _D)
    a2 = ln2_bias.reshape(1, _D)

    h1 = _mm1(embed.astype(jnp.float32), W1, b1r)
    p1 = _spmm_sc(h1.reshape(2 * _N, _DH), row, colA, colB, vals, zeros_nd)
    e1, h2 = _mid(p1, W2, b2r, g1, a1)
    p2 = _spmm_sc(h2.reshape(2 * _N, _DH), row, colA, colB, vals, zeros_nd)
    t = _fin(p2, e1, g2, a2)
    final = _lookup_sc(t, x.astype(jnp.int32))
    recon_loss = jnp.zeros((1,), dtype=jnp.float32)
    return (final, recon_loss)


# speedup vs baseline: 8.6811x; 1.1169x over previous
import functools

import jax
import jax.numpy as jnp
from jax import lax
from jax.experimental import pallas as pl
from jax.experimental.pallas import tpu as pltpu
from jax.experimental.pallas import tpu_sc as plsc

_N = 10000
_E = 320000
_D = 128
_B = 16384

_NC = 2
_NS = 16
_L = 16
_NW = _NC * _NS
_DH = _D // 2
_EPT = _E // _NS
_K = 80
_NCHUNK = _EPT // _K
_RPT = 624
_RTAIL = _N - _NS * _RPT

_sc_mesh = plsc.VectorSubcoreMesh(core_axis_name="c", subcore_axis_name="s")


_NBUF = 5
_W = 3


@functools.partial(
    pl.kernel,
    out_type=jax.ShapeDtypeStruct((_NC, _N, _DH), jnp.float32),
    mesh=_sc_mesh,
    scratch_types=[
        pltpu.VMEM((_NCHUNK, _K), jnp.int32),
        pltpu.VMEM((_NCHUNK, _K), jnp.int32),
        pltpu.VMEM((_NCHUNK, _K), jnp.float32),
        pltpu.VMEM((_NBUF, _K, _DH), jnp.float32),
        pltpu.VMEM_SHARED((_N, _DH), jnp.float32),
        pltpu.SemaphoreType.DMA((_NBUF,)),
        pltpu.SemaphoreType.DMA((_NBUF,)),
    ],
    compiler_params=pltpu.CompilerParams(use_tc_tiling_on_sc=False),
)
def _spmm_sc(e2_hbm, row_hbm, colA_hbm, colB_hbm, val_hbm, zeros_hbm, out_hbm,
             colv, rowv, valv, rows, acc, gsem, ssem):
    s = lax.axis_index("s")
    c = lax.axis_index("c")
    @pl.when(c == 0)
    def _():
        pltpu.sync_copy(colA_hbm.at[s], colv)

    @pl.when(c == 1)
    def _():
        pltpu.sync_copy(colB_hbm.at[s], colv)

    pltpu.sync_copy(row_hbm.at[s], rowv)
    pltpu.sync_copy(val_hbm.at[s], valv)
    for b in range(_W):
        pltpu.async_copy(e2_hbm.at[colv.at[b]], rows.at[b], gsem.at[b])
    pltpu.sync_copy(zeros_hbm.at[pl.ds(s * _RPT, _RPT)],
                    acc.at[pl.ds(s * _RPT, _RPT)])

    @pl.when(s == 0)
    def _():
        pltpu.sync_copy(zeros_hbm.at[pl.ds(_NS * _RPT, _RTAIL)],
                        acc.at[pl.ds(_NS * _RPT, _RTAIL)])

    plsc.subcore_barrier()

    def outer_body(jo, carry):
        for b in range(_NBUF):
            j = jo * _NBUF + b
            bn = (b + _W) % _NBUF

            @pl.when(j + _W >= _NBUF)
            def _():
                jp = j + _W - _NBUF
                pltpu.make_async_copy(rows.at[bn], acc.at[rowv.at[jp]],
                                      ssem.at[bn]).wait()

            @pl.when(j + _W < _NCHUNK)
            def _():
                pltpu.async_copy(e2_hbm.at[colv.at[j + _W]], rows.at[bn],
                                 gsem.at[bn])

            pltpu.make_async_copy(e2_hbm.at[colv.at[j]], rows.at[b],
                                  gsem.at[b]).wait()
            for g in range(_K // _L):
                v16 = valv[j, pl.ds(g * _L, _L)]
                for ll in range(_L):
                    sp = jnp.broadcast_to(v16[ll], (_L,))
                    e_i = g * _L + ll
                    for d in range(_DH // _L):
                        sl = pl.ds(d * _L, _L)
                        rows[b, e_i, sl] = rows[b, e_i, sl] * sp
            pltpu.async_copy(rows.at[b], acc.at[rowv.at[j]], ssem.at[b],
                             add=True)

        return carry

    lax.fori_loop(0, _NCHUNK // _NBUF, outer_body, 0)
    for jp in range(_NCHUNK - (_NBUF - _W), _NCHUNK):
        pltpu.make_async_copy(rows.at[jp % _NBUF], acc.at[rowv.at[jp]],
                              ssem.at[jp % _NBUF]).wait()
    plsc.subcore_barrier()
    pltpu.sync_copy(acc.at[pl.ds(s * _RPT, _RPT)],
                    out_hbm.at[c, pl.ds(s * _RPT, _RPT)])

    @pl.when(s == 0)
    def _():
        pltpu.sync_copy(acc.at[pl.ds(_NS * _RPT, _RTAIL)],
                        out_hbm.at[c, pl.ds(_NS * _RPT, _RTAIL)])


_BPT = _B // _NW
_GK = 128
_GCH = _BPT // _GK


@functools.partial(
    pl.kernel,
    out_type=jax.ShapeDtypeStruct((_B, _D), jnp.float32),
    mesh=_sc_mesh,
    scratch_types=[
        pltpu.VMEM((_GK,), jnp.int32),
        pltpu.VMEM((_GK,), jnp.int32),
        pltpu.VMEM((_GK, _D), jnp.float32),
        pltpu.SemaphoreType.DMA,
    ],
)
def _lookup_sc(t_hbm, x_hbm, out_hbm, xv, idxv, rows, sem):
    s = lax.axis_index("s")
    c = lax.axis_index("c")
    w = s * _NC + c
    base = w * _BPT

    def body(j, carry):
        off = base + j * _GK
        pltpu.sync_copy(x_hbm.at[pl.ds(off, _GK)], xv)
        for g in range(_GK // _L):
            sl = pl.ds(g * _L, _L)
            x16 = xv[sl]
            idxv[sl] = jnp.where(x16 >= 1, x16 - 1, _N)
        pltpu.async_copy(t_hbm.at[idxv], rows, sem).wait()
        pltpu.sync_copy(rows, out_hbm.at[pl.ds(off, _GK)])
        return carry

    lax.fori_loop(0, _GCH, body, 0)


def _mm1_body(e_ref, w_ref, b_ref, o_ref):
    o_ref[...] = (jnp.dot(e_ref[...], w_ref[...],
                          preferred_element_type=jnp.float32) + b_ref[...])


_mm1 = pl.pallas_call(
    _mm1_body,
    out_shape=jax.ShapeDtypeStruct((_N, _D), jnp.float32),
)


def _layer_norm(r, g, b):
    mu = jnp.mean(r, axis=-1, keepdims=True)
    var = jnp.mean((r - mu) ** 2, axis=-1, keepdims=True)
    return (r - mu) * lax.rsqrt(var + 1e-5) * g + b


def _mid_body(p_ref, w_ref, b_ref, g_ref, be_ref, e1_ref, h2_ref):
    r = jnp.maximum(jnp.concatenate([p_ref[0], p_ref[1]], axis=-1), 0.0)
    e1 = _layer_norm(r, g_ref[...], be_ref[...])
    e1_ref[...] = e1
    h2_ref[...] = (jnp.dot(e1, w_ref[...],
                           preferred_element_type=jnp.float32) + b_ref[...])


_mid = pl.pallas_call(
    _mid_body,
    out_shape=(jax.ShapeDtypeStruct((_N, _D), jnp.float32),
               jax.ShapeDtypeStruct((_N, _D), jnp.float32)),
)


def _fin_body(p_ref, e1_ref, g_ref, be_ref, o_ref):
    r = jnp.maximum(jnp.concatenate([p_ref[0], p_ref[1]], axis=-1), 0.0)
    e = _layer_norm(r, g_ref[...], be_ref[...]) + e1_ref[...]
    o_ref[pl.ds(0, _N), :] = e
    o_ref[pl.ds(_N, 8), :] = jnp.zeros((8, _D), jnp.float32)


_fin = pl.pallas_call(
    _fin_body,
    out_shape=jax.ShapeDtypeStruct((_N + 8, _D), jnp.float32),
)


def kernel(x, embed, G_indices, G_values, W1, b1, W2, b2,
           ln1_scale, ln1_bias, ln2_scale, ln2_bias):
    row = G_indices[0].reshape(_NS, _NCHUNK, _K)
    colA = (G_indices[1] * 2).reshape(_NS, _NCHUNK, _K)
    colB = (G_indices[1] * 2 + 1).reshape(_NS, _NCHUNK, _K)
    vals = G_values.reshape(_NS, _NCHUNK, _K)
    zeros_nd = jnp.zeros((_N, _DH), jnp.float32)
    b1r = b1.reshape(1, _D)
    b2r = b2.reshape(1, _D)
    g1 = ln1_scale.reshape(1, _D)
    a1 = ln1_bias.reshape(1, _D)
    g2 = ln2_scale.reshape(1, _D)
    a2 = ln2_bias.reshape(1, _D)

    h1 = _mm1(embed.astype(jnp.float32), W1, b1r)
    p1 = _spmm_sc(h1.reshape(2 * _N, _DH), row, colA, colB, vals, zeros_nd)
    e1, h2 = _mid(p1, W2, b2r, g1, a1)
    p2 = _spmm_sc(h2.reshape(2 * _N, _DH), row, colA, colB, vals, zeros_nd)
    t = _fin(p2, e1, g2, a2)
    final = _lookup_sc(t, x.astype(jnp.int32))
    recon_loss = jnp.zeros((1,), dtype=jnp.float32)
    return (final, recon_loss)
